# Initial kernel scaffold; baseline (speedup 1.0000x reference)
import jax
import jax.numpy as jnp
from jax.experimental import pallas as pl


def kernel(x, adj, W1, b1, W2, b2):
    raise NotImplementedError("write your pallas kernel here")


# speedup vs baseline: 4.8069x; 4.8069x over previous
import functools

import jax
import jax.numpy as jnp
from jax import lax
from jax.experimental import pallas as pl
from jax.experimental.pallas import tpu as pltpu
from jax.experimental.pallas import tpu_sc as plsc

_N = 10000
_NPAD = 10240
_NW = 32
_CHUNK = 128
_STRIPE = _NPAD // 16


def _make_agg(K: int, D: int):
  mesh = plsc.VectorSubcoreMesh(core_axis_name="c", subcore_axis_name="s")

  @functools.partial(
      pl.kernel,
      out_type=jax.ShapeDtypeStruct((2, _NPAD, D), jnp.float32),
      mesh=mesh,
      scratch_types=[
          pltpu.VMEM((K, _CHUNK), jnp.int32),
          pltpu.VMEM((K, _CHUNK), jnp.int32),
          pltpu.VMEM((_CHUNK, D), jnp.float32),
          pltpu.VMEM_SHARED((_NPAD, D), jnp.float32),
          pltpu.SemaphoreType.DMA,
      ],
      compiler_params=pltpu.CompilerParams(use_tc_tiling_on_sc=False),
  )
  def agg(x_hbm, src_hbm, dst_hbm, out_hbm, src_v, dst_v, rows_v, acc, sem):
    c = lax.axis_index("c")
    s = lax.axis_index("s")
    wid = s * 2 + c

    zeros16 = jnp.zeros((16,), jnp.float32)
    nwords = D // 16

    def zbody(i, carry):
      r = i // nwords
      col = (i % nwords) * 16
      rows_v[r, pl.ds(col, 16)] = zeros16
      return carry

    lax.fori_loop(0, _CHUNK * nwords, zbody, 0)
    base = s * _STRIPE
    for i in range(_STRIPE // _CHUNK):
      pltpu.sync_copy(rows_v, acc.at[pl.ds(base + i * _CHUNK, _CHUNK)])
    plsc.subcore_barrier()

    pltpu.sync_copy(src_hbm.at[wid], src_v)
    pltpu.sync_copy(dst_hbm.at[wid], dst_v)

    def body(j, carry):
      pltpu.async_copy(x_hbm.at[src_v.at[j]], rows_v, sem).wait()
      pltpu.sync_copy(rows_v, acc.at[dst_v.at[j]], add=True)
      return carry

    lax.fori_loop(0, K, body, 0)
    plsc.subcore_barrier()

    pltpu.sync_copy(acc.at[pl.ds(base, _STRIPE)],
                    out_hbm.at[c, pl.ds(base, _STRIPE)])

  return agg


def _mid_body(p0_ref, p1_ref, w1_ref, b1_ref, w2_ref, o_ref):
  agg = p0_ref[0] + p1_ref[0]
  h = jnp.dot(agg, w1_ref[...], preferred_element_type=jnp.float32)
  h = jnp.maximum(h + b1_ref[...], 0.0)
  o_ref[...] = jnp.dot(h, w2_ref[...], preferred_element_type=jnp.float32)


def _fin_body(q0_ref, q1_ref, b2_ref, o_ref):
  z = q0_ref[0] + q1_ref[0] + b2_ref[...]
  m = jnp.max(z, axis=1, keepdims=True)
  e = jnp.exp(z - m)
  ssum = jnp.sum(e, axis=1, keepdims=True)
  o_ref[...] = (z - m) - jnp.log(ssum)


def kernel(x, adj, W1, b1, W2, b2):
  n, d1 = x.shape
  d2 = W2.shape[1]
  e = adj.shape[1]
  k = -(-e // (_NW * _CHUNK))
  epad = _NW * k * _CHUNK

  adj = adj.astype(jnp.int32)
  src = jnp.concatenate([adj[0], jnp.zeros((epad - e,), jnp.int32)])
  dst = jnp.concatenate([adj[1], jnp.full((epad - e,), _N, jnp.int32)])
  src = src.reshape(_NW, k, _CHUNK)
  dst = dst.reshape(_NW, k, _CHUNK)

  p = _make_agg(k, d1)(x, src, dst)

  rb = 1000
  grid = n // rb
  t = pl.pallas_call(
      _mid_body,
      grid=(grid,),
      in_specs=[
          pl.BlockSpec((1, rb, d1), lambda i: (0, i, 0)),
          pl.BlockSpec((1, rb, d1), lambda i: (1, i, 0)),
          pl.BlockSpec((d1, d1), lambda i: (0, 0)),
          pl.BlockSpec((1, d1), lambda i: (0, 0)),
          pl.BlockSpec((d1, d2), lambda i: (0, 0)),
      ],
      out_specs=pl.BlockSpec((rb, d2), lambda i: (i, 0)),
      out_shape=jax.ShapeDtypeStruct((n, d2), jnp.float32),
  )(p, p, W1, b1.reshape(1, d1), W2)

  q = _make_agg(k, d2)(t, src, dst)

  out = pl.pallas_call(
      _fin_body,
      grid=(grid,),
      in_specs=[
          pl.BlockSpec((1, rb, d2), lambda i: (0, i, 0)),
          pl.BlockSpec((1, rb, d2), lambda i: (1, i, 0)),
          pl.BlockSpec((1, d2), lambda i: (0, 0)),
      ],
      out_specs=pl.BlockSpec((rb, d2), lambda i: (i, 0)),
      out_shape=jax.ShapeDtypeStruct((n, d2), jnp.float32),
  )(q, q, b2.reshape(1, d2))
  return out
